# Initial kernel scaffold; baseline (speedup 1.0000x reference)
import jax
import jax.numpy as jnp
from jax.experimental import pallas as pl


def kernel(x, edge_index, edge_attr, W_ee, b_ee, W_pre, b_pre, W_post, b_post, W_lin, b_lin, bn_gamma, bn_beta):
    raise NotImplementedError("write your pallas kernel here")


# speedup vs baseline: 1.9392x; 1.9392x over previous
import functools

import numpy as np
import jax
import jax.numpy as jnp
from jax import lax
from jax.experimental import pallas as pl
from jax.experimental.pallas import tpu as pltpu
from jax.experimental.pallas import tpu_sc as plsc

N = 10000
E = 320000
F = 128
DE = 16
L = 2
AVG_DEG_LOG = float(np.log(33.0))

NR = 64
RN = 160
NPAD = NR * RN
CH = 128
EP = E + CH
NB = 512
GN = NPAD // NB
EB = 4000
GE = E // EB

_f32 = jnp.float32


def _xdxs_body(x_ref, w_ref, xd_ref, xs_ref):
    xw = jnp.dot(x_ref[...], w_ref[...], preferred_element_type=_f32)
    xd_ref[...] = xw[:, :F]
    xs_ref[...] = xw[:, F:]


def _tc_xdxs(xp, wds):
    return pl.pallas_call(
        _xdxs_body,
        grid=(GN,),
        in_specs=[
            pl.BlockSpec((NB, F), lambda i: (i, 0)),
            pl.BlockSpec((F, 2 * F), lambda i: (0, 0)),
        ],
        out_specs=[
            pl.BlockSpec((NB, F), lambda i: (i, 0)),
            pl.BlockSpec((NB, F), lambda i: (i, 0)),
        ],
        out_shape=[
            jax.ShapeDtypeStruct((NPAD, F), _f32),
            jax.ShapeDtypeStruct((NPAD, F), _f32),
        ],
    )(xp, wds)


def _ee_body(ea_ref, wee_ref, bee_ref, we_ref, bpre_ref, out_ref):
    wc = jnp.dot(wee_ref[...], we_ref[...], preferred_element_type=_f32)
    bc = jnp.dot(bee_ref[...], we_ref[...], preferred_element_type=_f32) + bpre_ref[...]
    out_ref[...] = jnp.dot(ea_ref[...], wc, preferred_element_type=_f32) + bc


def _tc_ee(ea, wee, bee, we, bpre):
    return pl.pallas_call(
        _ee_body,
        grid=(GE,),
        in_specs=[
            pl.BlockSpec((EB, DE), lambda i: (i, 0)),
            pl.BlockSpec((DE, F), lambda i: (0, 0)),
            pl.BlockSpec((1, F), lambda i: (0, 0)),
            pl.BlockSpec((F, F), lambda i: (0, 0)),
            pl.BlockSpec((1, F), lambda i: (0, 0)),
        ],
        out_specs=pl.BlockSpec((EB, F), lambda i: (i, 0)),
        out_shape=jax.ShapeDtypeStruct((E, F), _f32),
    )(ea, wee, bee, we, bpre)


def _sc_body(xs_hbm, ee_hbm, srcp_hbm, ordp_hbm, dstp_hbm, meta_hbm,
             outS, outQ, outMN, outMX, outDC,
             meta_v, idxs_v, idxo_v, dstl_v, xsb, eeb,
             accS, accQ, accMN, accMX, accC, sem):
    wid = lax.axis_index("s") * 2 + lax.axis_index("c")
    pltpu.sync_copy(meta_hbm, meta_v)
    zero16 = jnp.zeros((16,), _f32)
    pinf16 = jnp.full((16,), jnp.inf, _f32)
    ninf16 = jnp.full((16,), -jnp.inf, _f32)
    one16 = jnp.full((16,), 1.0, _f32)
    mv = meta_v[pl.ds(wid * 16, 16)]
    for p in range(2):
        r = wid * 2 + p
        base = r * RN
        ws_r = mv[2 * p]
        nk_r = mv[2 * p + 1]

        def _init(i, c):
            for j in range(8):
                sl = pl.ds(j * 16, 16)
                accS[i, sl] = zero16
                accQ[i, sl] = zero16
                accMN[i, sl] = pinf16
                accMX[i, sl] = ninf16
            accC[pl.ds(pl.multiple_of(i * 16, 8), 16)] = zero16
            return c

        lax.fori_loop(0, RN + 1, _init, 0)

        def _chunk(k, c):
            st = pl.multiple_of(ws_r + k * CH, CH)
            pltpu.sync_copy(srcp_hbm.at[pl.ds(st, CH)], idxs_v)
            pltpu.sync_copy(ordp_hbm.at[pl.ds(st, CH)], idxo_v)
            pltpu.sync_copy(dstp_hbm.at[pl.ds(st, CH)], dstl_v)
            cpa = pltpu.async_copy(xs_hbm.at[idxs_v], xsb, sem)
            cpb = pltpu.async_copy(ee_hbm.at[idxo_v], eeb, sem)
            for j in range(8):
                sl = pl.ds(j * 16, 16)
                dv = dstl_v[sl] - base
                bad = (dv < 0) | (dv >= RN)
                dstl_v[sl] = jnp.where(bad, RN, dv)
            cpa.wait()
            cpb.wait()

            def _grp(g, cc):
                dv = dstl_v[pl.ds(pl.multiple_of(g * 16, 8), 16)]
                for kk in range(16):
                    d = dv[kk]
                    e = g * 16 + kk
                    co = pl.ds(pl.multiple_of(d * 16, 8), 16)
                    accC[co] = accC[co] + one16
                    for j in range(8):
                        sl = pl.ds(j * 16, 16)
                        b = xsb[e, sl] + eeb[e, sl]
                        accS[d, sl] = accS[d, sl] + b
                        accQ[d, sl] = accQ[d, sl] + b * b
                        accMN[d, sl] = jnp.minimum(accMN[d, sl], b)
                        accMX[d, sl] = jnp.maximum(accMX[d, sl], b)
                return cc

            lax.fori_loop(0, CH // 16, _grp, 0)
            return c

        lax.fori_loop(0, nk_r, _chunk, 0)
        rows = pl.ds(0, RN)
        orow = pl.ds(base, RN)
        pltpu.sync_copy(accS.at[rows], outS.at[orow])
        pltpu.sync_copy(accQ.at[rows], outQ.at[orow])
        pltpu.sync_copy(accMN.at[rows], outMN.at[orow])
        pltpu.sync_copy(accMX.at[rows], outMX.at[orow])
        pltpu.sync_copy(accC.at[pl.ds(0, RN * 16)],
                        outDC.at[pl.ds(pl.multiple_of(base * 16, 8), RN * 16)])


@functools.cache
def _sc_reduce_fn():
    return functools.partial(
        pl.kernel,
        out_type=[
            jax.ShapeDtypeStruct((NPAD, F), _f32),
            jax.ShapeDtypeStruct((NPAD, F), _f32),
            jax.ShapeDtypeStruct((NPAD, F), _f32),
            jax.ShapeDtypeStruct((NPAD, F), _f32),
            jax.ShapeDtypeStruct((NPAD * 16,), _f32),
        ],
        mesh=plsc.VectorSubcoreMesh(core_axis_name="c", subcore_axis_name="s"),
        scratch_types=[
            pltpu.VMEM((32 * 16,), jnp.int32),
            pltpu.VMEM((CH,), jnp.int32),
            pltpu.VMEM((CH,), jnp.int32),
            pltpu.VMEM((CH,), jnp.int32),
            pltpu.VMEM((CH, F), _f32),
            pltpu.VMEM((CH, F), _f32),
            pltpu.VMEM((RN + 1, F), _f32),
            pltpu.VMEM((RN + 1, F), _f32),
            pltpu.VMEM((RN + 1, F), _f32),
            pltpu.VMEM((RN + 1, F), _f32),
            pltpu.VMEM(((RN + 1) * 16,), _f32),
            pltpu.SemaphoreType.DMA,
        ],
    )(_sc_body)


def _sc_reduce(*args):
    return _sc_reduce_fn()(*args)


def _comb_body(x_ref, xd_ref, s_ref, q_ref, mn_ref, mx_ref, dc_ref,
               wpost_ref, bpost_ref, wlin_ref, blin_ref, z_ref, st_ref):
    i = pl.program_id(0)
    deg = dc_ref[:, 0:1]
    degc = jnp.maximum(deg, 1.0)
    logd = jnp.log(degc + 1.0)
    amp = logd * (1.0 / AVG_DEG_LOG)
    att = AVG_DEG_LOG / logd
    has = deg > 0.0
    a = xd_ref[...]
    Sb = s_ref[...]
    Qb = q_ref[...]
    s = deg * a + Sb
    mean = s / degc
    msq = (deg * a * a + 2.0 * a * Sb + Qb) / degc
    std = jnp.sqrt(jnp.maximum(msq - mean * mean, 0.0) + 1e-5)
    mn = jnp.where(has, a + mn_ref[...], 0.0)
    mx = jnp.where(has, a + mx_ref[...], 0.0)
    aggr = [mean, mn, mx, std, s]
    parts = [x_ref[...]] + aggr + [g * amp for g in aggr] + [g * att for g in aggr]
    u = jnp.concatenate(parts, axis=1)
    z1 = jnp.dot(u, wpost_ref[...], preferred_element_type=_f32) + bpost_ref[...]
    z = jnp.dot(z1, wlin_ref[...], preferred_element_type=_f32) + blin_ref[...]
    z_ref[...] = z
    row = i * NB + lax.broadcasted_iota(jnp.int32, (NB, 1), 0)
    zm = jnp.where(row < N, z, 0.0)
    ps = jnp.sum(zm, axis=0, keepdims=True)
    psq = jnp.sum(zm * zm, axis=0, keepdims=True)
    acc = jnp.concatenate([ps, psq, jnp.zeros((6, F), _f32)], axis=0)

    @pl.when(i == 0)
    def _():
        st_ref[...] = acc

    @pl.when(i > 0)
    def _():
        st_ref[...] = st_ref[...] + acc


def _tc_combine(xp, xd, S, Q, MN, MX, DC, wpost, bpost, wlin, blin):
    nspec = pl.BlockSpec((NB, F), lambda i: (i, 0))
    return pl.pallas_call(
        _comb_body,
        grid=(GN,),
        in_specs=[
            nspec, nspec, nspec, nspec, nspec, nspec,
            pl.BlockSpec((NB, 16), lambda i: (i, 0)),
            pl.BlockSpec((16 * F, F), lambda i: (0, 0)),
            pl.BlockSpec((1, F), lambda i: (0, 0)),
            pl.BlockSpec((F, F), lambda i: (0, 0)),
            pl.BlockSpec((1, F), lambda i: (0, 0)),
        ],
        out_specs=[
            pl.BlockSpec((NB, F), lambda i: (i, 0)),
            pl.BlockSpec((8, F), lambda i: (0, 0)),
        ],
        out_shape=[
            jax.ShapeDtypeStruct((NPAD, F), _f32),
            jax.ShapeDtypeStruct((8, F), _f32),
        ],
    )(xp, xd, S, Q, MN, MX, DC, wpost, bpost, wlin, blin)


def _bn_body(z_ref, st_ref, g_ref, b_ref, out_ref):
    i = pl.program_id(0)
    mu = st_ref[0:1, :] * (1.0 / N)
    var = st_ref[1:2, :] * (1.0 / N) - mu * mu
    inv = g_ref[...] * lax.rsqrt(var + 1e-5)
    z = (z_ref[...] - mu) * inv + b_ref[...]
    row = i * NB + lax.broadcasted_iota(jnp.int32, (NB, 1), 0)
    out_ref[...] = jnp.where(row < N, jnp.maximum(z, 0.0), 0.0)


def _tc_bnrelu(z, st, gamma, beta):
    return pl.pallas_call(
        _bn_body,
        grid=(GN,),
        in_specs=[
            pl.BlockSpec((NB, F), lambda i: (i, 0)),
            pl.BlockSpec((8, F), lambda i: (0, 0)),
            pl.BlockSpec((1, F), lambda i: (0, 0)),
            pl.BlockSpec((1, F), lambda i: (0, 0)),
        ],
        out_specs=pl.BlockSpec((NB, F), lambda i: (i, 0)),
        out_shape=jax.ShapeDtypeStruct((NPAD, F), _f32),
    )(z, st, gamma, beta)


def kernel(x, edge_index, edge_attr, W_ee, b_ee, W_pre, b_pre, W_post, b_post,
           W_lin, b_lin, bn_gamma, bn_beta):
    src = edge_index[0]
    dst = edge_index[1]
    order = jnp.argsort(dst).astype(jnp.int32)
    dst_s = jnp.take(dst, order)
    src_s = jnp.take(src, order)
    bounds = (jnp.arange(NR + 1, dtype=jnp.int32) * RN)
    offs = jnp.searchsorted(dst_s, bounds).astype(jnp.int32)
    ws = (offs[:NR] // CH) * CH
    nk = (offs[1:] - ws + (CH - 1)) // CH
    wsnk = jnp.stack([ws, nk], axis=1).reshape(32, 4)
    meta = jnp.pad(wsnk, ((0, 0), (0, 12))).reshape(512).astype(jnp.int32)
    srcp = jnp.concatenate([src_s, jnp.zeros((CH,), jnp.int32)])
    ordp = jnp.concatenate([order, jnp.zeros((CH,), jnp.int32)])
    dstp = jnp.concatenate([dst_s, jnp.full((CH,), NPAD, jnp.int32)])
    xp = jnp.pad(x, ((0, NPAD - N), (0, 0)))

    for l in range(L):
        wds = jnp.concatenate([W_pre[l, :F, :], W_pre[l, F:2 * F, :]], axis=1)
        we = W_pre[l, 2 * F:, :]
        xd, xs = _tc_xdxs(xp, wds)
        ee = _tc_ee(edge_attr, W_ee[l], b_ee[l].reshape(1, F), we,
                    b_pre[l].reshape(1, F))
        S, Q, MN, MX, DC = _sc_reduce(xs, ee, srcp, ordp, dstp, meta)
        DC = DC.reshape(NPAD, 16)
        z, st = _tc_combine(xp, xd, S, Q, MN, MX, DC, W_post[l],
                            b_post[l].reshape(1, F), W_lin[l],
                            b_lin[l].reshape(1, F))
        xp = _tc_bnrelu(z, st, bn_gamma[l].reshape(1, F),
                        bn_beta[l].reshape(1, F))
    return xp[:N]


# speedup vs baseline: 4.7525x; 2.4508x over previous
import functools

import numpy as np
import jax
import jax.numpy as jnp
from jax import lax
from jax.experimental import pallas as pl
from jax.experimental.pallas import tpu as pltpu
from jax.experimental.pallas import tpu_sc as plsc

N = 10000
E = 320000
F = 128
DE = 16
L = 2
AVG_DEG_LOG = float(np.log(33.0))

NR = 64
RN = 160
NPAD = NR * RN
CH = 128
EP = E + CH
NB = 512
GN = NPAD // NB
EB = 4000
GE = E // EB

_f32 = jnp.float32


def _xdxs_body(x_ref, w_ref, xd_ref, xs_ref):
    xw = jnp.dot(x_ref[...], w_ref[...], preferred_element_type=_f32)
    xd_ref[...] = xw[:, :F]
    xs_ref[...] = xw[:, F:]


def _tc_xdxs(xp, wds):
    return pl.pallas_call(
        _xdxs_body,
        grid=(GN,),
        in_specs=[
            pl.BlockSpec((NB, F), lambda i: (i, 0)),
            pl.BlockSpec((F, 2 * F), lambda i: (0, 0)),
        ],
        out_specs=[
            pl.BlockSpec((NB, F), lambda i: (i, 0)),
            pl.BlockSpec((NB, F), lambda i: (i, 0)),
        ],
        out_shape=[
            jax.ShapeDtypeStruct((NPAD, F), _f32),
            jax.ShapeDtypeStruct((NPAD, F), _f32),
        ],
    )(xp, wds)


def _ee_body(ea_ref, wee_ref, bee_ref, we_ref, bpre_ref, out_ref):
    wc = jnp.dot(wee_ref[...], we_ref[...], preferred_element_type=_f32)
    bc = jnp.dot(bee_ref[...], we_ref[...], preferred_element_type=_f32) + bpre_ref[...]
    out_ref[...] = jnp.dot(ea_ref[...], wc, preferred_element_type=_f32) + bc


def _tc_ee(ea, wee, bee, we, bpre):
    return pl.pallas_call(
        _ee_body,
        grid=(GE,),
        in_specs=[
            pl.BlockSpec((EB, DE), lambda i: (i, 0)),
            pl.BlockSpec((DE, F), lambda i: (0, 0)),
            pl.BlockSpec((1, F), lambda i: (0, 0)),
            pl.BlockSpec((F, F), lambda i: (0, 0)),
            pl.BlockSpec((1, F), lambda i: (0, 0)),
        ],
        out_specs=pl.BlockSpec((EB, F), lambda i: (i, 0)),
        out_shape=jax.ShapeDtypeStruct((E, F), _f32),
    )(ea, wee, bee, we, bpre)


def _sc_body(xs_hbm, ee_hbm, srcp_hbm, ordp_hbm, dstp_hbm, meta_hbm,
             outS, outQ, outMN, outMX, outDC,
             meta_v, idxs_v, idxo_v, dstl_v, xsb, eeb,
             accS, accQ, accMN, accMX, accC, sem):
    wid = lax.axis_index("s") * 2 + lax.axis_index("c")
    pltpu.sync_copy(meta_hbm, meta_v)
    zero16 = jnp.zeros((16,), _f32)
    pinf16 = jnp.full((16,), jnp.inf, _f32)
    ninf16 = jnp.full((16,), -jnp.inf, _f32)
    one16 = jnp.full((16,), 1.0, _f32)
    mv = meta_v[pl.ds(wid * 16, 16)]
    for p in range(2):
        r = wid * 2 + p
        base = r * RN
        ws_r = mv[2 * p]
        nk_r = mv[2 * p + 1]

        def _init(i, c):
            ro = pl.multiple_of(i * F, 8)
            for j in range(8):
                sl = pl.ds(ro + j * 16, 16)
                accS[sl] = zero16
                accQ[sl] = zero16
                accMN[sl] = pinf16
                accMX[sl] = ninf16
            accC[pl.ds(pl.multiple_of(i * 16, 8), 16)] = zero16
            return c

        lax.fori_loop(0, RN + 1, _init, 0)

        def _flush(dc, ss, qq, mn, mx, cv):
            ro = pl.multiple_of(dc * F, 8)
            for j in range(8):
                sl = pl.ds(ro + j * 16, 16)
                accS[sl] = ss[j]
                accQ[sl] = qq[j]
                accMN[sl] = mn[j]
                accMX[sl] = mx[j]
            accC[pl.ds(pl.multiple_of(dc * 16, 8), 16)] = cv

        def _chunk(k, carry):
            st = pl.multiple_of(ws_r + k * CH, CH)
            pltpu.sync_copy(srcp_hbm.at[pl.ds(st, CH)], idxs_v)
            pltpu.sync_copy(ordp_hbm.at[pl.ds(st, CH)], idxo_v)
            pltpu.sync_copy(dstp_hbm.at[pl.ds(st, CH)], dstl_v)
            cpa = pltpu.async_copy(xs_hbm.at[idxs_v], xsb, sem)
            cpb = pltpu.async_copy(ee_hbm.at[idxo_v], eeb, sem)
            for j in range(8):
                sl = pl.ds(j * 16, 16)
                dv = dstl_v[sl] - base
                bad = (dv < 0) | (dv >= RN)
                dstl_v[sl] = jnp.where(bad, RN, dv)
            cpa.wait()
            cpb.wait()

            def _grp(g, gc):
                dv = dstl_v[pl.ds(pl.multiple_of(g * 16, 8), 16)]
                ss, qq, mn, mx, cv, dc = gc
                for kk in range(16):
                    d = dv[kk]
                    e = g * 16 + kk
                    c = d != dc

                    @pl.when(c)
                    def _():
                        _flush(dc, ss, qq, mn, mx, cv)

                    b = [xsb[e, pl.ds(j * 16, 16)] + eeb[e, pl.ds(j * 16, 16)]
                         for j in range(8)]
                    ss = tuple(jnp.where(c, b[j], ss[j] + b[j]) for j in range(8))
                    qq = tuple(jnp.where(c, b[j] * b[j], qq[j] + b[j] * b[j])
                               for j in range(8))
                    mn = tuple(jnp.where(c, b[j], jnp.minimum(mn[j], b[j]))
                               for j in range(8))
                    mx = tuple(jnp.where(c, b[j], jnp.maximum(mx[j], b[j]))
                               for j in range(8))
                    cv = jnp.where(c, one16, cv + one16)
                    dc = d
                return (ss, qq, mn, mx, cv, dc)

            return lax.fori_loop(0, CH // 16, _grp, carry)

        carry0 = (
            tuple(zero16 for _ in range(8)),
            tuple(zero16 for _ in range(8)),
            tuple(pinf16 for _ in range(8)),
            tuple(ninf16 for _ in range(8)),
            zero16,
            jnp.int32(RN),
        )
        ss, qq, mn, mx, cv, dc = lax.fori_loop(0, nk_r, _chunk, carry0)
        _flush(dc, ss, qq, mn, mx, cv)
        rows = pl.ds(0, RN * F)
        orow = pl.ds(pl.multiple_of(base * F, 8), RN * F)
        pltpu.sync_copy(accS.at[rows], outS.at[orow])
        pltpu.sync_copy(accQ.at[rows], outQ.at[orow])
        pltpu.sync_copy(accMN.at[rows], outMN.at[orow])
        pltpu.sync_copy(accMX.at[rows], outMX.at[orow])
        pltpu.sync_copy(accC.at[pl.ds(0, RN * 16)],
                        outDC.at[pl.ds(pl.multiple_of(base * 16, 8), RN * 16)])


@functools.cache
def _sc_reduce_fn():
    return functools.partial(
        pl.kernel,
        out_type=[
            jax.ShapeDtypeStruct((NPAD * F,), _f32),
            jax.ShapeDtypeStruct((NPAD * F,), _f32),
            jax.ShapeDtypeStruct((NPAD * F,), _f32),
            jax.ShapeDtypeStruct((NPAD * F,), _f32),
            jax.ShapeDtypeStruct((NPAD * 16,), _f32),
        ],
        mesh=plsc.VectorSubcoreMesh(core_axis_name="c", subcore_axis_name="s"),
        scratch_types=[
            pltpu.VMEM((32 * 16,), jnp.int32),
            pltpu.VMEM((CH,), jnp.int32),
            pltpu.VMEM((CH,), jnp.int32),
            pltpu.VMEM((CH,), jnp.int32),
            pltpu.VMEM((CH, F), _f32),
            pltpu.VMEM((CH, F), _f32),
            pltpu.VMEM(((RN + 1) * F,), _f32),
            pltpu.VMEM(((RN + 1) * F,), _f32),
            pltpu.VMEM(((RN + 1) * F,), _f32),
            pltpu.VMEM(((RN + 1) * F,), _f32),
            pltpu.VMEM(((RN + 1) * 16,), _f32),
            pltpu.SemaphoreType.DMA,
        ],
    )(_sc_body)


def _sc_reduce(*args):
    return _sc_reduce_fn()(*args)


def _comb_body(x_ref, xd_ref, s_ref, q_ref, mn_ref, mx_ref, dc_ref,
               wpost_ref, bpost_ref, wlin_ref, blin_ref, z_ref, st_ref):
    i = pl.program_id(0)
    deg = dc_ref[:, 0:1]
    degc = jnp.maximum(deg, 1.0)
    logd = jnp.log(degc + 1.0)
    amp = logd * (1.0 / AVG_DEG_LOG)
    att = AVG_DEG_LOG / logd
    has = deg > 0.0
    a = xd_ref[...]
    Sb = s_ref[...]
    Qb = q_ref[...]
    s = deg * a + Sb
    mean = s / degc
    msq = (deg * a * a + 2.0 * a * Sb + Qb) / degc
    std = jnp.sqrt(jnp.maximum(msq - mean * mean, 0.0) + 1e-5)
    mn = jnp.where(has, a + mn_ref[...], 0.0)
    mx = jnp.where(has, a + mx_ref[...], 0.0)
    aggr = [mean, mn, mx, std, s]
    parts = [x_ref[...]] + aggr + [g * amp for g in aggr] + [g * att for g in aggr]
    u = jnp.concatenate(parts, axis=1)
    z1 = jnp.dot(u, wpost_ref[...], preferred_element_type=_f32) + bpost_ref[...]
    z = jnp.dot(z1, wlin_ref[...], preferred_element_type=_f32) + blin_ref[...]
    z_ref[...] = z
    row = i * NB + lax.broadcasted_iota(jnp.int32, (NB, 1), 0)
    zm = jnp.where(row < N, z, 0.0)
    ps = jnp.sum(zm, axis=0, keepdims=True)
    psq = jnp.sum(zm * zm, axis=0, keepdims=True)
    acc = jnp.concatenate([ps, psq, jnp.zeros((6, F), _f32)], axis=0)

    @pl.when(i == 0)
    def _():
        st_ref[...] = acc

    @pl.when(i > 0)
    def _():
        st_ref[...] = st_ref[...] + acc


def _tc_combine(xp, xd, S, Q, MN, MX, DC, wpost, bpost, wlin, blin):
    nspec = pl.BlockSpec((NB, F), lambda i: (i, 0))
    return pl.pallas_call(
        _comb_body,
        grid=(GN,),
        in_specs=[
            nspec, nspec, nspec, nspec, nspec, nspec,
            pl.BlockSpec((NB, 16), lambda i: (i, 0)),
            pl.BlockSpec((16 * F, F), lambda i: (0, 0)),
            pl.BlockSpec((1, F), lambda i: (0, 0)),
            pl.BlockSpec((F, F), lambda i: (0, 0)),
            pl.BlockSpec((1, F), lambda i: (0, 0)),
        ],
        out_specs=[
            pl.BlockSpec((NB, F), lambda i: (i, 0)),
            pl.BlockSpec((8, F), lambda i: (0, 0)),
        ],
        out_shape=[
            jax.ShapeDtypeStruct((NPAD, F), _f32),
            jax.ShapeDtypeStruct((8, F), _f32),
        ],
    )(xp, xd, S, Q, MN, MX, DC, wpost, bpost, wlin, blin)


def _bn_body(z_ref, st_ref, g_ref, b_ref, out_ref):
    i = pl.program_id(0)
    mu = st_ref[0:1, :] * (1.0 / N)
    var = st_ref[1:2, :] * (1.0 / N) - mu * mu
    inv = g_ref[...] * lax.rsqrt(var + 1e-5)
    z = (z_ref[...] - mu) * inv + b_ref[...]
    row = i * NB + lax.broadcasted_iota(jnp.int32, (NB, 1), 0)
    out_ref[...] = jnp.where(row < N, jnp.maximum(z, 0.0), 0.0)


def _tc_bnrelu(z, st, gamma, beta):
    return pl.pallas_call(
        _bn_body,
        grid=(GN,),
        in_specs=[
            pl.BlockSpec((NB, F), lambda i: (i, 0)),
            pl.BlockSpec((8, F), lambda i: (0, 0)),
            pl.BlockSpec((1, F), lambda i: (0, 0)),
            pl.BlockSpec((1, F), lambda i: (0, 0)),
        ],
        out_specs=pl.BlockSpec((NB, F), lambda i: (i, 0)),
        out_shape=jax.ShapeDtypeStruct((NPAD, F), _f32),
    )(z, st, gamma, beta)


def kernel(x, edge_index, edge_attr, W_ee, b_ee, W_pre, b_pre, W_post, b_post,
           W_lin, b_lin, bn_gamma, bn_beta):
    src = edge_index[0]
    dst = edge_index[1]
    order = jnp.argsort(dst).astype(jnp.int32)
    dst_s = jnp.take(dst, order)
    src_s = jnp.take(src, order)
    bounds = (jnp.arange(NR + 1, dtype=jnp.int32) * RN)
    offs = jnp.searchsorted(dst_s, bounds).astype(jnp.int32)
    ws = (offs[:NR] // CH) * CH
    nk = (offs[1:] - ws + (CH - 1)) // CH
    wsnk = jnp.stack([ws, nk], axis=1).reshape(32, 4)
    meta = jnp.pad(wsnk, ((0, 0), (0, 12))).reshape(512).astype(jnp.int32)
    srcp = jnp.concatenate([src_s, jnp.zeros((CH,), jnp.int32)])
    ordp = jnp.concatenate([order, jnp.zeros((CH,), jnp.int32)])
    dstp = jnp.concatenate([dst_s, jnp.full((CH,), NPAD, jnp.int32)])
    xp = jnp.pad(x, ((0, NPAD - N), (0, 0)))

    for l in range(L):
        wds = jnp.concatenate([W_pre[l, :F, :], W_pre[l, F:2 * F, :]], axis=1)
        we = W_pre[l, 2 * F:, :]
        xd, xs = _tc_xdxs(xp, wds)
        ee = _tc_ee(edge_attr, W_ee[l], b_ee[l].reshape(1, F), we,
                    b_pre[l].reshape(1, F))
        S, Q, MN, MX, DC = _sc_reduce(xs, ee, srcp, ordp, dstp, meta)
        S, Q, MN, MX = (a.reshape(NPAD, F) for a in (S, Q, MN, MX))
        DC = DC.reshape(NPAD, 16)
        z, st = _tc_combine(xp, xd, S, Q, MN, MX, DC, W_post[l],
                            b_post[l].reshape(1, F), W_lin[l],
                            b_lin[l].reshape(1, F))
        xp = _tc_bnrelu(z, st, bn_gamma[l].reshape(1, F),
                        bn_beta[l].reshape(1, F))
    return xp[:N]


# speedup vs baseline: 5.2721x; 1.1093x over previous
import functools

import numpy as np
import jax
import jax.numpy as jnp
from jax import lax
from jax.experimental import pallas as pl
from jax.experimental.pallas import tpu as pltpu
from jax.experimental.pallas import tpu_sc as plsc

N = 10000
E = 320000
F = 128
DE = 16
L = 2
AVG_DEG_LOG = float(np.log(33.0))

NR = 64
RN = 160
NPAD = NR * RN
CH = 64
SUP = 512
CPS = SUP // CH
EP = E + SUP
NB = 512
GN = NPAD // NB
EB = 4000
GE = E // EB

_f32 = jnp.float32


def _xdxs_body(x_ref, w_ref, xd_ref, xs_ref):
    xw = jnp.dot(x_ref[...], w_ref[...], preferred_element_type=_f32)
    xd_ref[...] = xw[:, :F]
    xs_ref[...] = xw[:, F:]


def _tc_xdxs(xp, wds):
    return pl.pallas_call(
        _xdxs_body,
        grid=(GN,),
        in_specs=[
            pl.BlockSpec((NB, F), lambda i: (i, 0)),
            pl.BlockSpec((F, 2 * F), lambda i: (0, 0)),
        ],
        out_specs=[
            pl.BlockSpec((NB, F), lambda i: (i, 0)),
            pl.BlockSpec((NB, F), lambda i: (i, 0)),
        ],
        out_shape=[
            jax.ShapeDtypeStruct((NPAD, F), _f32),
            jax.ShapeDtypeStruct((NPAD, F), _f32),
        ],
    )(xp, wds)


def _ee_body(ea_ref, wee_ref, bee_ref, we_ref, bpre_ref, out_ref):
    wc = jnp.dot(wee_ref[...], we_ref[...], preferred_element_type=_f32)
    bc = jnp.dot(bee_ref[...], we_ref[...], preferred_element_type=_f32) + bpre_ref[...]
    out_ref[...] = jnp.dot(ea_ref[...], wc, preferred_element_type=_f32) + bc


def _tc_ee(ea, wee, bee, we, bpre):
    return pl.pallas_call(
        _ee_body,
        grid=(GE,),
        in_specs=[
            pl.BlockSpec((EB, DE), lambda i: (i, 0)),
            pl.BlockSpec((DE, F), lambda i: (0, 0)),
            pl.BlockSpec((1, F), lambda i: (0, 0)),
            pl.BlockSpec((F, F), lambda i: (0, 0)),
            pl.BlockSpec((1, F), lambda i: (0, 0)),
        ],
        out_specs=pl.BlockSpec((EB, F), lambda i: (i, 0)),
        out_shape=jax.ShapeDtypeStruct((E, F), _f32),
    )(ea, wee, bee, we, bpre)


def _sc_body(xs_hbm, ee_hbm, srcp_hbm, ordp_hbm, dstp_hbm, meta_hbm,
             outS, outQ, outMN, outMX, outDC,
             meta_v, idxs_v, idxo_v, idxd_v, xsb0, xsb1, eeb0, eeb1,
             accS, accQ, accMN, accMX, accC, semA, semB):
    wid = lax.axis_index("s") * 2 + lax.axis_index("c")
    pltpu.sync_copy(meta_hbm, meta_v)
    zero16 = jnp.zeros((16,), _f32)
    pinf16 = jnp.full((16,), jnp.inf, _f32)
    ninf16 = jnp.full((16,), -jnp.inf, _f32)
    one16 = jnp.full((16,), 1.0, _f32)
    mv = meta_v[pl.ds(wid * 16, 16)]
    bufs = ((xsb0, eeb0, semA), (xsb1, eeb1, semB))
    for t in range(CH // 16):
        tl = pl.ds(SUP + t * 16, 16)
        spread = wid * CH + t * 16 + lax.iota(jnp.int32, 16)
        idxs_v[tl] = spread
        idxo_v[tl] = spread
    for p in range(2):
        r = wid * 2 + p
        base = r * RN
        ws_r = mv[2 * p]
        ns_r = mv[2 * p + 1]

        def _init(i, c):
            ro = pl.multiple_of(i * F, 8)
            for j in range(8):
                sl = pl.ds(ro + j * 16, 16)
                accS[sl] = zero16
                accQ[sl] = zero16
                accMN[sl] = pinf16
                accMX[sl] = ninf16
            accC[pl.ds(pl.multiple_of(i * 16, 8), 16)] = zero16
            return c

        lax.fori_loop(0, RN + 1, _init, 0)

        def _flush(dc, ss, qq, mn, mx, cv):
            ro = pl.multiple_of(dc * F, 8)
            for j in range(8):
                sl = pl.ds(ro + j * 16, 16)
                accS[sl] = ss[j]
                accQ[sl] = qq[j]
                accMN[sl] = mn[j]
                accMX[sl] = mx[j]
            accC[pl.ds(pl.multiple_of(dc * 16, 8), 16)] = cv

        def _process(xb, eb, off, carry):
            def _grp(g, gc):
                dv = idxd_v[pl.ds(pl.multiple_of(g * 16 + off, 8), 16)]
                ss, qq, mn, mx, cv, dc = gc
                for kk in range(16):
                    d = dv[kk]
                    e = g * 16 + kk
                    c = d != dc

                    @pl.when(c)
                    def _():
                        _flush(dc, ss, qq, mn, mx, cv)

                    b = [xb[e, pl.ds(j * 16, 16)] + eb[e, pl.ds(j * 16, 16)]
                         for j in range(8)]
                    ss = tuple(jnp.where(c, b[j], ss[j] + b[j]) for j in range(8))
                    qq = tuple(jnp.where(c, b[j] * b[j], qq[j] + b[j] * b[j])
                               for j in range(8))
                    mn = tuple(jnp.where(c, b[j], jnp.minimum(mn[j], b[j]))
                               for j in range(8))
                    mx = tuple(jnp.where(c, b[j], jnp.maximum(mx[j], b[j]))
                               for j in range(8))
                    cv = jnp.where(c, one16, cv + one16)
                    dc = d
                return (ss, qq, mn, mx, cv, dc)

            return lax.fori_loop(0, CH // 16, _grp, carry)

        def _issue(buf, off):
            xb, eb, sm = buf
            o = pl.multiple_of(off, 8)
            cpa = pltpu.async_copy(
                xs_hbm.at[idxs_v.at[pl.ds(o, CH)]], xb, sm)
            cpb = pltpu.async_copy(
                ee_hbm.at[idxo_v.at[pl.ds(o, CH)]], eb, sm)
            return cpa, cpb

        def _wait(buf):
            xb, eb, sm = buf
            pltpu.make_async_copy(
                xs_hbm.at[idxs_v.at[pl.ds(0, CH)]], xb, sm).wait()
            pltpu.make_async_copy(
                ee_hbm.at[idxo_v.at[pl.ds(0, CH)]], eb, sm).wait()

        def _sup(s, carry):
            stu = pl.multiple_of(ws_r + s * SUP, CH)
            pltpu.sync_copy(srcp_hbm.at[pl.ds(stu, SUP)], idxs_v.at[pl.ds(0, SUP)])
            pltpu.sync_copy(ordp_hbm.at[pl.ds(stu, SUP)], idxo_v.at[pl.ds(0, SUP)])
            pltpu.sync_copy(dstp_hbm.at[pl.ds(stu, SUP)], idxd_v)

            def _tr(t, c):
                sl = pl.ds(pl.multiple_of(t * 16, 8), 16)
                dvv = idxd_v[sl] - base
                bad = (dvv < 0) | (dvv >= RN)
                idxd_v[sl] = jnp.where(bad, RN, dvv)
                return c

            lax.fori_loop(0, SUP // 16, _tr, 0)
            _issue(bufs[0], 0)

            def _pair(q, cc):
                _issue(bufs[1], (2 * q + 1) * CH)
                _wait(bufs[0])
                cc = _process(bufs[0][0], bufs[0][1], 2 * q * CH, cc)
                _issue(bufs[0], (2 * q + 2) * CH)
                _wait(bufs[1])
                cc = _process(bufs[1][0], bufs[1][1], (2 * q + 1) * CH, cc)
                return cc

            carry = lax.fori_loop(0, CPS // 2, _pair, carry)
            _wait(bufs[0])
            return carry

        carry0 = (
            tuple(zero16 for _ in range(8)),
            tuple(zero16 for _ in range(8)),
            tuple(pinf16 for _ in range(8)),
            tuple(ninf16 for _ in range(8)),
            zero16,
            jnp.int32(RN),
        )
        ss, qq, mn, mx, cv, dc = lax.fori_loop(0, ns_r, _sup, carry0)
        _flush(dc, ss, qq, mn, mx, cv)
        rows = pl.ds(0, RN * F)
        orow = pl.ds(pl.multiple_of(base * F, 8), RN * F)
        pltpu.sync_copy(accS.at[rows], outS.at[orow])
        pltpu.sync_copy(accQ.at[rows], outQ.at[orow])
        pltpu.sync_copy(accMN.at[rows], outMN.at[orow])
        pltpu.sync_copy(accMX.at[rows], outMX.at[orow])
        pltpu.sync_copy(accC.at[pl.ds(0, RN * 16)],
                        outDC.at[pl.ds(pl.multiple_of(base * 16, 8), RN * 16)])


@functools.cache
def _sc_reduce_fn():
    return functools.partial(
        pl.kernel,
        out_type=[
            jax.ShapeDtypeStruct((NPAD * F,), _f32),
            jax.ShapeDtypeStruct((NPAD * F,), _f32),
            jax.ShapeDtypeStruct((NPAD * F,), _f32),
            jax.ShapeDtypeStruct((NPAD * F,), _f32),
            jax.ShapeDtypeStruct((NPAD * 16,), _f32),
        ],
        mesh=plsc.VectorSubcoreMesh(core_axis_name="c", subcore_axis_name="s"),
        scratch_types=[
            pltpu.VMEM((32 * 16,), jnp.int32),
            pltpu.VMEM((SUP + CH,), jnp.int32),
            pltpu.VMEM((SUP + CH,), jnp.int32),
            pltpu.VMEM((SUP,), jnp.int32),
            pltpu.VMEM((CH, F), _f32),
            pltpu.VMEM((CH, F), _f32),
            pltpu.VMEM((CH, F), _f32),
            pltpu.VMEM((CH, F), _f32),
            pltpu.VMEM(((RN + 1) * F,), _f32),
            pltpu.VMEM(((RN + 1) * F,), _f32),
            pltpu.VMEM(((RN + 1) * F,), _f32),
            pltpu.VMEM(((RN + 1) * F,), _f32),
            pltpu.VMEM(((RN + 1) * 16,), _f32),
            pltpu.SemaphoreType.DMA,
            pltpu.SemaphoreType.DMA,
        ],
    )(_sc_body)


def _sc_reduce(*args):
    return _sc_reduce_fn()(*args)


def _comb_body(x_ref, xd_ref, s_ref, q_ref, mn_ref, mx_ref, dc_ref,
               wpost_ref, bpost_ref, wlin_ref, blin_ref, z_ref, st_ref):
    i = pl.program_id(0)
    deg = dc_ref[:, 0:1]
    degc = jnp.maximum(deg, 1.0)
    logd = jnp.log(degc + 1.0)
    amp = logd * (1.0 / AVG_DEG_LOG)
    att = AVG_DEG_LOG / logd
    has = deg > 0.0
    a = xd_ref[...]
    Sb = s_ref[...]
    Qb = q_ref[...]
    s = deg * a + Sb
    mean = s / degc
    msq = (deg * a * a + 2.0 * a * Sb + Qb) / degc
    std = jnp.sqrt(jnp.maximum(msq - mean * mean, 0.0) + 1e-5)
    mn = jnp.where(has, a + mn_ref[...], 0.0)
    mx = jnp.where(has, a + mx_ref[...], 0.0)
    aggr = [mean, mn, mx, std, s]
    parts = [x_ref[...]] + aggr + [g * amp for g in aggr] + [g * att for g in aggr]
    u = jnp.concatenate(parts, axis=1)
    z1 = jnp.dot(u, wpost_ref[...], preferred_element_type=_f32) + bpost_ref[...]
    z = jnp.dot(z1, wlin_ref[...], preferred_element_type=_f32) + blin_ref[...]
    z_ref[...] = z
    row = i * NB + lax.broadcasted_iota(jnp.int32, (NB, 1), 0)
    zm = jnp.where(row < N, z, 0.0)
    ps = jnp.sum(zm, axis=0, keepdims=True)
    psq = jnp.sum(zm * zm, axis=0, keepdims=True)
    acc = jnp.concatenate([ps, psq, jnp.zeros((6, F), _f32)], axis=0)

    @pl.when(i == 0)
    def _():
        st_ref[...] = acc

    @pl.when(i > 0)
    def _():
        st_ref[...] = st_ref[...] + acc


def _tc_combine(xp, xd, S, Q, MN, MX, DC, wpost, bpost, wlin, blin):
    nspec = pl.BlockSpec((NB, F), lambda i: (i, 0))
    return pl.pallas_call(
        _comb_body,
        grid=(GN,),
        in_specs=[
            nspec, nspec, nspec, nspec, nspec, nspec,
            pl.BlockSpec((NB, 16), lambda i: (i, 0)),
            pl.BlockSpec((16 * F, F), lambda i: (0, 0)),
            pl.BlockSpec((1, F), lambda i: (0, 0)),
            pl.BlockSpec((F, F), lambda i: (0, 0)),
            pl.BlockSpec((1, F), lambda i: (0, 0)),
        ],
        out_specs=[
            pl.BlockSpec((NB, F), lambda i: (i, 0)),
            pl.BlockSpec((8, F), lambda i: (0, 0)),
        ],
        out_shape=[
            jax.ShapeDtypeStruct((NPAD, F), _f32),
            jax.ShapeDtypeStruct((8, F), _f32),
        ],
    )(xp, xd, S, Q, MN, MX, DC, wpost, bpost, wlin, blin)


def _bn_body(z_ref, st_ref, g_ref, b_ref, out_ref):
    i = pl.program_id(0)
    mu = st_ref[0:1, :] * (1.0 / N)
    var = st_ref[1:2, :] * (1.0 / N) - mu * mu
    inv = g_ref[...] * lax.rsqrt(var + 1e-5)
    z = (z_ref[...] - mu) * inv + b_ref[...]
    row = i * NB + lax.broadcasted_iota(jnp.int32, (NB, 1), 0)
    out_ref[...] = jnp.where(row < N, jnp.maximum(z, 0.0), 0.0)


def _tc_bnrelu(z, st, gamma, beta):
    return pl.pallas_call(
        _bn_body,
        grid=(GN,),
        in_specs=[
            pl.BlockSpec((NB, F), lambda i: (i, 0)),
            pl.BlockSpec((8, F), lambda i: (0, 0)),
            pl.BlockSpec((1, F), lambda i: (0, 0)),
            pl.BlockSpec((1, F), lambda i: (0, 0)),
        ],
        out_specs=pl.BlockSpec((NB, F), lambda i: (i, 0)),
        out_shape=jax.ShapeDtypeStruct((NPAD, F), _f32),
    )(z, st, gamma, beta)


def kernel(x, edge_index, edge_attr, W_ee, b_ee, W_pre, b_pre, W_post, b_post,
           W_lin, b_lin, bn_gamma, bn_beta):
    src = edge_index[0]
    dst = edge_index[1]
    order = jnp.argsort(dst).astype(jnp.int32)
    dst_s = jnp.take(dst, order)
    src_s = jnp.take(src, order)
    bounds = (jnp.arange(NR + 1, dtype=jnp.int32) * RN)
    offs = jnp.searchsorted(dst_s, bounds).astype(jnp.int32)
    ws = (offs[:NR] // CH) * CH
    ns = (offs[1:] - ws + (SUP - 1)) // SUP
    wsns = jnp.stack([ws, ns], axis=1).reshape(32, 4)
    meta = jnp.pad(wsns, ((0, 0), (0, 12))).reshape(512).astype(jnp.int32)
    srcp = jnp.concatenate([src_s, jnp.zeros((SUP,), jnp.int32)])
    ordp = jnp.concatenate([order, jnp.zeros((SUP,), jnp.int32)])
    dstp = jnp.concatenate([dst_s, jnp.full((SUP,), NPAD, jnp.int32)])
    xp = jnp.pad(x, ((0, NPAD - N), (0, 0)))

    for l in range(L):
        wds = jnp.concatenate([W_pre[l, :F, :], W_pre[l, F:2 * F, :]], axis=1)
        we = W_pre[l, 2 * F:, :]
        xd, xs = _tc_xdxs(xp, wds)
        ee = _tc_ee(edge_attr, W_ee[l], b_ee[l].reshape(1, F), we,
                    b_pre[l].reshape(1, F))
        S, Q, MN, MX, DC = _sc_reduce(xs, ee, srcp, ordp, dstp, meta)
        S, Q, MN, MX = (a.reshape(NPAD, F) for a in (S, Q, MN, MX))
        DC = DC.reshape(NPAD, 16)
        z, st = _tc_combine(xp, xd, S, Q, MN, MX, DC, W_post[l],
                            b_post[l].reshape(1, F), W_lin[l],
                            b_lin[l].reshape(1, F))
        xp = _tc_bnrelu(z, st, bn_gamma[l].reshape(1, F),
                        bn_beta[l].reshape(1, F))
    return xp[:N]
